# baseline (speedup 1.0000x reference)
import jax
import jax.numpy as jnp
from jax.experimental import pallas as pl

_F32 = jnp.float32
_LO = (-51.2, -51.2, -5.0)
_HI = (51.2, 51.2, 3.0)
_EPS = 1e-6
_ALPHA = 0.1
_NP = 4
_FH, _FW = 16, 44
_NPIX = _FH * _FW
_NCAM = 6
_H = _W = 100
_Q = _H * _W


def _conv5_res_body(pref, wref, bref, oref):
    y = pl.program_id(0)
    rows = pref[pl.ds(y, 5)]
    acc = rows[2][2:2 + _W]
    acc = acc + bref[...]
    for dy in range(5):
        r = rows[dy]
        a = jnp.concatenate([r[dx:dx + _W] for dx in range(5)], axis=1)
        acc = acc + jnp.dot(a, wref[dy], preferred_element_type=_F32)
    oref[0] = acc


def _conv5_res(p, w25, b):
    return pl.pallas_call(
        _conv5_res_body,
        grid=(_H,),
        in_specs=[
            pl.BlockSpec(p.shape, lambda y: (0, 0, 0)),
            pl.BlockSpec(w25.shape, lambda y: (0, 0, 0)),
            pl.BlockSpec(b.shape, lambda y: (0, 0)),
        ],
        out_specs=pl.BlockSpec((1, _W, 128), lambda y: (y, 0, 0)),
        out_shape=jax.ShapeDtypeStruct((_H, _W, 128), _F32),
    )(p, w25, b)


def _in_body(xref, oref):
    x = xref[...]
    m = jnp.mean(x, axis=0, keepdims=True)
    d = x - m
    v = jnp.mean(d * d, axis=0, keepdims=True)
    oref[...] = d * jax.lax.rsqrt(v + 1e-5)


def _inorm(x):
    return pl.pallas_call(
        _in_body,
        out_shape=jax.ShapeDtypeStruct(x.shape, _F32),
    )(x)


def _add_in_body(aref, bref, oref):
    x = aref[...] + bref[...]
    m = jnp.mean(x, axis=0, keepdims=True)
    d = x - m
    v = jnp.mean(d * d, axis=0, keepdims=True)
    oref[...] = d * jax.lax.rsqrt(v + 1e-5)


def _add_inorm(a, b):
    return pl.pallas_call(
        _add_in_body,
        out_shape=jax.ShapeDtypeStruct(a.shape, _F32),
    )(a, b)


def _samp_body(qref, posref, fref, lref, offtref, ofbref, projtref, pjbref,
               postref, psbref, sfref, sfwref):
    G = _W
    span = (_HI[0] - _LO[0], _HI[1] - _LO[1], _HI[2] - _LO[2])
    q = qref[0]
    offm = jax.nn.sigmoid(jnp.dot(q, offtref[...],
                                  preferred_element_type=_F32) + ofbref[...])
    pos = posref[0]
    refm = jnp.concatenate(
        [pos[:, i:i + 1] * span[i] + _LO[i] for i in range(3)], axis=1)

    iof = jax.lax.broadcasted_iota(jnp.int32, (_NP * G, _NPIX), 1).astype(_F32)
    pyf = jnp.floor(iof * (1.0 / _FW))
    pxf = iof - pyf * _FW

    rng = 0.25 + _EPS
    rngz = 4.0 + _EPS
    refp = []
    for p in range(_NP):
        o = offm[:, 3 * p:3 * p + 3]
        oxy = o[:, 0:2] * (2.0 * rng) - rng
        oz = o[:, 2:3] * (2.0 * rngz) - rngz
        refp.append(refm + jnp.concatenate([oxy, oz], axis=1))
    rp = jnp.concatenate(refp, axis=0)
    R = _NP * G

    samp = jnp.zeros((R, 132), _F32)
    ones = jnp.ones((R, 1), _F32)
    for cam in range(_NCAM):
        lt = lref[cam]
        fcam = fref[cam]
        px = (rp[:, 0:1] * lt[0, 0] + rp[:, 1:2] * lt[1, 0]
              + rp[:, 2:3] * lt[2, 0] + lt[3, 0] * ones)
        py = (rp[:, 0:1] * lt[0, 1] + rp[:, 1:2] * lt[1, 1]
              + rp[:, 2:3] * lt[2, 1] + lt[3, 1] * ones)
        pz = (rp[:, 0:1] * lt[0, 2] + rp[:, 1:2] * lt[1, 2]
              + rp[:, 2:3] * lt[2, 2] + lt[3, 2] * ones)
        vz = (pz > 1e-5).astype(_F32)
        zc = jnp.maximum(pz, 1e-5)
        u = px / zc / 704.0 * _FW - 0.5
        v = py / zc / 256.0 * _FH - 0.5
        x0 = jnp.floor(u)
        y0 = jnp.floor(v)
        x1 = x0 + 1.0
        y1 = y0 + 1.0
        wx1 = u - x0
        wx0 = 1.0 - wx1
        wy1 = v - y0
        wy0 = 1.0 - wy1

        def oneh(grid_f, coord, wgt, lim):
            okc = ((coord >= 0.0) & (coord <= lim)).astype(_F32) * wgt
            eq = (grid_f == jnp.clip(coord, 0.0, lim)).astype(_F32)
            return eq * okc

        ax = (oneh(pxf, x0, wx0, _FW - 1.0)
              + oneh(pxf, x1, wx1, _FW - 1.0))
        ay = (oneh(pyf, y0, wy0, _FH - 1.0)
              + oneh(pyf, y1, wy1, _FH - 1.0))
        wcp = ay * ax * vz
        samp = samp + jnp.dot(wcp, fcam, preferred_element_type=_F32)

    projt = projtref[...]
    pjb = pjbref[...]
    post = postref[...]
    psb = psbref[...]
    sf_pre = jnp.dot(samp[:, :128], projt, preferred_element_type=_F32) + pjb
    pos3 = samp[:, 128:131]
    d = rp - pos3
    nrm = jnp.sqrt(jnp.sum(d * d, axis=1, keepdims=True))
    wgt = jnp.exp(-_ALPHA * nrm * nrm)
    rn = [(rp[:, i:i + 1] - _LO[i]) * (1.0 / span[i]) for i in range(3)]
    pe = (rn[0] * post[0:1, :] + rn[1] * post[1:2, :]
          + rn[2] * post[2:3, :] + psb)
    sfall = sf_pre + pe
    sfwall = sf_pre * wgt
    sfacc = sfall[0:G]
    sfwacc = sfwall[0:G]
    for p in range(1, _NP):
        sfacc = sfacc + sfall[p * G:(p + 1) * G]
        sfwacc = sfwacc + sfwall[p * G:(p + 1) * G]
    sfref[0] = sfacc
    sfwref[0] = sfwacc


def _sampling(qf, posf, fcat, l2t, offt, ofb, projt, pjb, post, psb):
    full = lambda a: pl.BlockSpec(a.shape, lambda y: (0,) * a.ndim)
    return pl.pallas_call(
        _samp_body,
        grid=(_H,),
        in_specs=[
            pl.BlockSpec((1, _W, 128), lambda y: (y, 0, 0)),
            pl.BlockSpec((1, _W, 3), lambda y: (y, 0, 0)),
            full(fcat), full(l2t), full(offt), full(ofb),
            full(projt), full(pjb), full(post), full(psb),
        ],
        out_specs=[
            pl.BlockSpec((1, _W, 128), lambda y: (y, 0, 0)),
            pl.BlockSpec((1, _W, 128), lambda y: (y, 0, 0)),
        ],
        out_shape=[
            jax.ShapeDtypeStruct((_H, _W, 128), _F32),
            jax.ShapeDtypeStruct((_H, _W, 128), _F32),
        ],
    )(qf, posf, fcat, l2t, offt, ofb, projt, pjb, post, psb)


def _mid12_body(pref, w1ref, b1ref, w2ref, b2ref, oref):
    y = pl.program_id(0)
    rows = pref[pl.ds(y, 3)]
    acc = jnp.zeros((_W, 512), _F32) + b1ref[...]
    for dy in range(3):
        r = rows[dy]
        a = jnp.concatenate([r[dx:dx + _W] for dx in range(3)], axis=1)
        acc = acc + jnp.dot(a, w1ref[dy], preferred_element_type=_F32)
    acc = jax.nn.gelu(acc)
    acc = jnp.dot(acc, w2ref[...], preferred_element_type=_F32) + b2ref[...]
    oref[0] = jax.nn.gelu(acc)


def _mid12(p, w1, b1, w2, b2):
    full = lambda a: pl.BlockSpec(a.shape, lambda y: (0,) * a.ndim)
    return pl.pallas_call(
        _mid12_body,
        grid=(_H,),
        in_specs=[full(p), full(w1), full(b1), full(w2), full(b2)],
        out_specs=pl.BlockSpec((1, _W, 512), lambda y: (y, 0, 0)),
        out_shape=jax.ShapeDtypeStruct((_H, _W, 512), _F32),
    )(p, w1, b1, w2, b2)


def _mid3_body(pref, wref, bref, oref):
    y = pl.program_id(0)
    rows = pref[pl.ds(y, 3)]
    acc = jnp.zeros((_W, 128), _F32) + bref[...]
    for dy in range(3):
        r = rows[dy]
        a = jnp.concatenate([r[dx:dx + _W] for dx in range(3)], axis=1)
        acc = acc + jnp.dot(a, wref[dy], preferred_element_type=_F32)
    oref[0] = acc


def _mid3(p, w3, b3):
    full = lambda a: pl.BlockSpec(a.shape, lambda y: (0,) * a.ndim)
    return pl.pallas_call(
        _mid3_body,
        grid=(_H,),
        in_specs=[full(p), full(w3), full(b3)],
        out_specs=pl.BlockSpec((1, _W, 128), lambda y: (y, 0, 0)),
        out_shape=jax.ShapeDtypeStruct((_H, _W, 128), _F32),
    )(p, w3, b3)


def kernel(bev_query, bev_pos, mlvl_feats, lidar2img, in_conv_w, in_conv_b,
           off_w, off_b, proj_w, proj_b, pos_w, pos_b, mid_w1, mid_b1,
           mid_w2, mid_b2, mid_w3, mid_b3, out_w, out_b):
    bq = jnp.transpose(bev_query[0], (1, 2, 0))
    p1 = jnp.pad(bq, ((2, 2), (2, 2), (0, 0)))
    w_in = jnp.transpose(in_conv_w, (2, 3, 1, 0)).reshape(5, 5 * 128, 128)
    r1 = _conv5_res(p1, w_in, in_conv_b[None])
    q1 = _inorm(r1.reshape(_Q, 128))

    fcat = jnp.transpose(mlvl_feats[0], (0, 2, 3, 1)).reshape(
        _NCAM, _NPIX, 132)
    l2t = jnp.transpose(lidar2img[0], (0, 2, 1))[:, :, :3]
    offt = off_w.reshape(12, 128).T
    sf, sfw = _sampling(q1.reshape(_H, _W, 128), bev_pos[0], fcat, l2t,
                        offt, off_b[None], proj_w.T, proj_b[None],
                        pos_w.T, pos_b[None])

    cat = jnp.concatenate([sf, sfw], axis=2)
    p2 = jnp.pad(cat, ((1, 1), (1, 1), (0, 0)))
    w1 = jnp.transpose(mid_w1, (2, 3, 1, 0)).reshape(3, 3 * 256, 512)
    w2 = mid_w2.reshape(512, 512).T
    mid2 = _mid12(p2, w1, mid_b1[None], w2, mid_b2[None])
    p3 = jnp.pad(mid2, ((1, 1), (1, 1), (0, 0)))
    w3 = jnp.transpose(mid_w3, (2, 3, 1, 0)).reshape(3, 3 * 512, 128)
    mid = _mid3(p3, w3, mid_b3[None])

    q2 = _add_inorm(q1, mid.reshape(_Q, 128))
    p4 = jnp.pad(q2.reshape(_H, _W, 128), ((2, 2), (2, 2), (0, 0)))
    w_out = jnp.transpose(out_w, (2, 3, 1, 0)).reshape(5, 5 * 128, 128)
    r2 = _conv5_res(p4, w_out, out_b[None])
    q3 = _inorm(r2.reshape(_Q, 128))
    return jnp.transpose(q3.reshape(_H, _W, 128), (2, 0, 1))[None]


# speedup vs baseline: 1.2914x; 1.2914x over previous
import jax
import jax.numpy as jnp
from jax.experimental import pallas as pl

_F32 = jnp.float32
_LO = (-51.2, -51.2, -5.0)
_HI = (51.2, 51.2, 3.0)
_EPS = 1e-6
_ALPHA = 0.1
_NP = 4
_FH, _FW = 16, 44
_NPIX = _FH * _FW
_NCAM = 6
_H = _W = 100
_Q = _H * _W


def _conv5_res_body(pref, wref, bref, oref):
    y = pl.program_id(0)
    rows = pref[pl.ds(y, 5)]
    acc = rows[2][2:2 + _W]
    acc = acc + bref[...]
    for dy in range(5):
        r = rows[dy]
        a = jnp.concatenate([r[dx:dx + _W] for dx in range(5)], axis=1)
        acc = acc + jnp.dot(a, wref[dy], preferred_element_type=_F32)
    oref[0] = acc


def _conv5_res(p, w25, b):
    return pl.pallas_call(
        _conv5_res_body,
        grid=(_H,),
        in_specs=[
            pl.BlockSpec(p.shape, lambda y: (0, 0, 0)),
            pl.BlockSpec(w25.shape, lambda y: (0, 0, 0)),
            pl.BlockSpec(b.shape, lambda y: (0, 0)),
        ],
        out_specs=pl.BlockSpec((1, _W, 128), lambda y: (y, 0, 0)),
        out_shape=jax.ShapeDtypeStruct((_H, _W, 128), _F32),
    )(p, w25, b)


def _in_body(xref, oref):
    x = xref[...]
    m = jnp.mean(x, axis=0, keepdims=True)
    d = x - m
    v = jnp.mean(d * d, axis=0, keepdims=True)
    oref[...] = d * jax.lax.rsqrt(v + 1e-5)


def _inorm(x):
    return pl.pallas_call(
        _in_body,
        out_shape=jax.ShapeDtypeStruct(x.shape, _F32),
    )(x)


def _add_in_body(aref, bref, oref):
    x = aref[...] + bref[...]
    m = jnp.mean(x, axis=0, keepdims=True)
    d = x - m
    v = jnp.mean(d * d, axis=0, keepdims=True)
    oref[...] = d * jax.lax.rsqrt(v + 1e-5)


def _add_inorm(a, b):
    return pl.pallas_call(
        _add_in_body,
        out_shape=jax.ShapeDtypeStruct(a.shape, _F32),
    )(a, b)


def _samp_body(qref, posref, fref, lref, offtref, ofbref, projtref, pjbref,
               postref, psbref, sfref, sfwref):
    G = _W
    span = (_HI[0] - _LO[0], _HI[1] - _LO[1], _HI[2] - _LO[2])
    q = qref[0]
    offm = jax.nn.sigmoid(jnp.dot(q, offtref[...],
                                  preferred_element_type=_F32) + ofbref[...])
    pos = posref[0]
    refm = jnp.concatenate(
        [pos[:, i:i + 1] * span[i] + _LO[i] for i in range(3)], axis=1)

    iof = jax.lax.broadcasted_iota(jnp.int32, (_NP * G, _NPIX), 1).astype(_F32)
    pyf = jnp.floor(iof * (1.0 / _FW))
    pxf = iof - pyf * _FW

    rng = 0.25 + _EPS
    rngz = 4.0 + _EPS
    refp = []
    for p in range(_NP):
        o = offm[:, 3 * p:3 * p + 3]
        oxy = o[:, 0:2] * (2.0 * rng) - rng
        oz = o[:, 2:3] * (2.0 * rngz) - rngz
        refp.append(refm + jnp.concatenate([oxy, oz], axis=1))
    rp = jnp.concatenate(refp, axis=0)
    R = _NP * G

    samp = jnp.zeros((R, 132), _F32)
    ones = jnp.ones((R, 1), _F32)
    for cam in range(_NCAM):
        lt = lref[cam]
        fcam = fref[cam]
        px = (rp[:, 0:1] * lt[0, 0] + rp[:, 1:2] * lt[1, 0]
              + rp[:, 2:3] * lt[2, 0] + lt[3, 0] * ones)
        py = (rp[:, 0:1] * lt[0, 1] + rp[:, 1:2] * lt[1, 1]
              + rp[:, 2:3] * lt[2, 1] + lt[3, 1] * ones)
        pz = (rp[:, 0:1] * lt[0, 2] + rp[:, 1:2] * lt[1, 2]
              + rp[:, 2:3] * lt[2, 2] + lt[3, 2] * ones)
        vz = (pz > 1e-5).astype(_F32)
        zc = jnp.maximum(pz, 1e-5)
        u = px / zc / 704.0 * _FW - 0.5
        v = py / zc / 256.0 * _FH - 0.5
        ax = jnp.maximum(1.0 - jnp.abs(u - pxf), 0.0)
        ay = jnp.maximum(1.0 - jnp.abs(v - pyf), 0.0)
        wcp = ay * ax * vz
        samp = samp + jnp.dot(wcp.astype(jnp.bfloat16),
                              fcam.astype(jnp.bfloat16),
                              preferred_element_type=_F32)

    projt = projtref[...]
    pjb = pjbref[...]
    post = postref[...]
    psb = psbref[...]
    sf_pre = jnp.dot(samp[:, :128], projt, preferred_element_type=_F32) + pjb
    pos3 = samp[:, 128:131]
    d = rp - pos3
    nrm = jnp.sqrt(jnp.sum(d * d, axis=1, keepdims=True))
    wgt = jnp.exp(-_ALPHA * nrm * nrm)
    rn = [(rp[:, i:i + 1] - _LO[i]) * (1.0 / span[i]) for i in range(3)]
    pe = (rn[0] * post[0:1, :] + rn[1] * post[1:2, :]
          + rn[2] * post[2:3, :] + psb)
    sfall = sf_pre + pe
    sfwall = sf_pre * wgt
    sfacc = sfall[0:G]
    sfwacc = sfwall[0:G]
    for p in range(1, _NP):
        sfacc = sfacc + sfall[p * G:(p + 1) * G]
        sfwacc = sfwacc + sfwall[p * G:(p + 1) * G]
    sfref[0] = sfacc
    sfwref[0] = sfwacc


def _sampling(qf, posf, fcat, l2t, offt, ofb, projt, pjb, post, psb):
    full = lambda a: pl.BlockSpec(a.shape, lambda y: (0,) * a.ndim)
    return pl.pallas_call(
        _samp_body,
        grid=(_H,),
        in_specs=[
            pl.BlockSpec((1, _W, 128), lambda y: (y, 0, 0)),
            pl.BlockSpec((1, _W, 3), lambda y: (y, 0, 0)),
            full(fcat), full(l2t), full(offt), full(ofb),
            full(projt), full(pjb), full(post), full(psb),
        ],
        out_specs=[
            pl.BlockSpec((1, _W, 128), lambda y: (y, 0, 0)),
            pl.BlockSpec((1, _W, 128), lambda y: (y, 0, 0)),
        ],
        out_shape=[
            jax.ShapeDtypeStruct((_H, _W, 128), _F32),
            jax.ShapeDtypeStruct((_H, _W, 128), _F32),
        ],
    )(qf, posf, fcat, l2t, offt, ofb, projt, pjb, post, psb)


def _mid12_body(pref, w1ref, b1ref, w2ref, b2ref, oref):
    y = pl.program_id(0)
    rows = pref[pl.ds(y, 3)]
    acc = jnp.zeros((_W, 512), _F32) + b1ref[...]
    for dy in range(3):
        r = rows[dy]
        a = jnp.concatenate([r[dx:dx + _W] for dx in range(3)], axis=1)
        acc = acc + jnp.dot(a, w1ref[dy], preferred_element_type=_F32)
    acc = jax.nn.gelu(acc)
    acc = jnp.dot(acc, w2ref[...], preferred_element_type=_F32) + b2ref[...]
    oref[0] = jax.nn.gelu(acc)


def _mid12(p, w1, b1, w2, b2):
    full = lambda a: pl.BlockSpec(a.shape, lambda y: (0,) * a.ndim)
    return pl.pallas_call(
        _mid12_body,
        grid=(_H,),
        in_specs=[full(p), full(w1), full(b1), full(w2), full(b2)],
        out_specs=pl.BlockSpec((1, _W, 512), lambda y: (y, 0, 0)),
        out_shape=jax.ShapeDtypeStruct((_H, _W, 512), _F32),
    )(p, w1, b1, w2, b2)


def _mid3_body(pref, wref, bref, oref):
    y = pl.program_id(0)
    rows = pref[pl.ds(y, 3)]
    acc = jnp.zeros((_W, 128), _F32) + bref[...]
    for dy in range(3):
        r = rows[dy]
        a = jnp.concatenate([r[dx:dx + _W] for dx in range(3)], axis=1)
        acc = acc + jnp.dot(a, wref[dy], preferred_element_type=_F32)
    oref[0] = acc


def _mid3(p, w3, b3):
    full = lambda a: pl.BlockSpec(a.shape, lambda y: (0,) * a.ndim)
    return pl.pallas_call(
        _mid3_body,
        grid=(_H,),
        in_specs=[full(p), full(w3), full(b3)],
        out_specs=pl.BlockSpec((1, _W, 128), lambda y: (y, 0, 0)),
        out_shape=jax.ShapeDtypeStruct((_H, _W, 128), _F32),
    )(p, w3, b3)


def kernel(bev_query, bev_pos, mlvl_feats, lidar2img, in_conv_w, in_conv_b,
           off_w, off_b, proj_w, proj_b, pos_w, pos_b, mid_w1, mid_b1,
           mid_w2, mid_b2, mid_w3, mid_b3, out_w, out_b):
    bq = jnp.transpose(bev_query[0], (1, 2, 0))
    p1 = jnp.pad(bq, ((2, 2), (2, 2), (0, 0)))
    w_in = jnp.transpose(in_conv_w, (2, 3, 1, 0)).reshape(5, 5 * 128, 128)
    r1 = _conv5_res(p1, w_in, in_conv_b[None])
    q1 = _inorm(r1.reshape(_Q, 128))

    fcat = jnp.transpose(mlvl_feats[0], (0, 2, 3, 1)).reshape(
        _NCAM, _NPIX, 132)
    l2t = jnp.transpose(lidar2img[0], (0, 2, 1))[:, :, :3]
    offt = off_w.reshape(12, 128).T
    sf, sfw = _sampling(q1.reshape(_H, _W, 128), bev_pos[0], fcat, l2t,
                        offt, off_b[None], proj_w.T, proj_b[None],
                        pos_w.T, pos_b[None])

    cat = jnp.concatenate([sf, sfw], axis=2)
    p2 = jnp.pad(cat, ((1, 1), (1, 1), (0, 0)))
    w1 = jnp.transpose(mid_w1, (2, 3, 1, 0)).reshape(3, 3 * 256, 512)
    w2 = mid_w2.reshape(512, 512).T
    mid2 = _mid12(p2, w1, mid_b1[None], w2, mid_b2[None])
    p3 = jnp.pad(mid2, ((1, 1), (1, 1), (0, 0)))
    w3 = jnp.transpose(mid_w3, (2, 3, 1, 0)).reshape(3, 3 * 512, 128)
    mid = _mid3(p3, w3, mid_b3[None])

    q2 = _add_inorm(q1, mid.reshape(_Q, 128))
    p4 = jnp.pad(q2.reshape(_H, _W, 128), ((2, 2), (2, 2), (0, 0)))
    w_out = jnp.transpose(out_w, (2, 3, 1, 0)).reshape(5, 5 * 128, 128)
    r2 = _conv5_res(p4, w_out, out_b[None])
    q3 = _inorm(r2.reshape(_Q, 128))
    return jnp.transpose(q3.reshape(_H, _W, 128), (2, 0, 1))[None]


# speedup vs baseline: 1.3565x; 1.0504x over previous
import jax
import jax.numpy as jnp
from jax.experimental import pallas as pl

_F32 = jnp.float32
_LO = (-51.2, -51.2, -5.0)
_HI = (51.2, 51.2, 3.0)
_EPS = 1e-6
_ALPHA = 0.1
_NP = 4
_FH, _FW = 16, 44
_NPIX = _FH * _FW
_NCAM = 6
_H = _W = 100
_Q = _H * _W


def _conv5_res_body(pref, wref, bref, oref):
    y = pl.program_id(0)
    rows = pref[pl.ds(y, 5)]
    acc = rows[2][2:2 + _W]
    acc = acc + bref[...]
    for dy in range(5):
        r = rows[dy]
        a = jnp.concatenate([r[dx:dx + _W] for dx in range(5)], axis=1)
        acc = acc + jnp.dot(a.astype(jnp.bfloat16), wref[dy],
                            preferred_element_type=_F32)
    oref[0] = acc


def _conv5_res(p, w25, b):
    return pl.pallas_call(
        _conv5_res_body,
        grid=(_H,),
        in_specs=[
            pl.BlockSpec(p.shape, lambda y: (0, 0, 0)),
            pl.BlockSpec(w25.shape, lambda y: (0, 0, 0)),
            pl.BlockSpec(b.shape, lambda y: (0, 0)),
        ],
        out_specs=pl.BlockSpec((1, _W, 128), lambda y: (y, 0, 0)),
        out_shape=jax.ShapeDtypeStruct((_H, _W, 128), _F32),
    )(p, w25, b)


def _in_body(xref, oref):
    x = xref[...]
    m = jnp.mean(x, axis=0, keepdims=True)
    d = x - m
    v = jnp.mean(d * d, axis=0, keepdims=True)
    oref[...] = d * jax.lax.rsqrt(v + 1e-5)


def _inorm(x):
    return pl.pallas_call(
        _in_body,
        out_shape=jax.ShapeDtypeStruct(x.shape, _F32),
    )(x)


def _add_in_body(aref, bref, oref):
    x = aref[...] + bref[...]
    m = jnp.mean(x, axis=0, keepdims=True)
    d = x - m
    v = jnp.mean(d * d, axis=0, keepdims=True)
    oref[...] = d * jax.lax.rsqrt(v + 1e-5)


def _add_inorm(a, b):
    return pl.pallas_call(
        _add_in_body,
        out_shape=jax.ShapeDtypeStruct(a.shape, _F32),
    )(a, b)


def _samp_body(qref, posref, fref, lref, offtref, ofbref, projtref, pjbref,
               postref, psbref, sfref, sfwref):
    G = _W
    span = (_HI[0] - _LO[0], _HI[1] - _LO[1], _HI[2] - _LO[2])
    q = qref[0]
    offm = jax.nn.sigmoid(jnp.dot(q, offtref[...],
                                  preferred_element_type=_F32) + ofbref[...])
    pos = posref[0]
    refm = jnp.concatenate(
        [pos[:, i:i + 1] * span[i] + _LO[i] for i in range(3)], axis=1)

    iof = jax.lax.broadcasted_iota(jnp.int32, (_NP * G, _NPIX), 1).astype(_F32)
    pyf = jnp.floor(iof * (1.0 / _FW))
    pxf = iof - pyf * _FW

    rng = 0.25 + _EPS
    rngz = 4.0 + _EPS
    refp = []
    for p in range(_NP):
        o = offm[:, 3 * p:3 * p + 3]
        oxy = o[:, 0:2] * (2.0 * rng) - rng
        oz = o[:, 2:3] * (2.0 * rngz) - rngz
        refp.append(refm + jnp.concatenate([oxy, oz], axis=1))
    rp = jnp.concatenate(refp, axis=0)
    R = _NP * G

    samp = jnp.zeros((R, 132), _F32)
    ones = jnp.ones((R, 1), _F32)
    for cam in range(_NCAM):
        lt = lref[cam]
        fcam = fref[cam]
        px = (rp[:, 0:1] * lt[0, 0] + rp[:, 1:2] * lt[1, 0]
              + rp[:, 2:3] * lt[2, 0] + lt[3, 0] * ones)
        py = (rp[:, 0:1] * lt[0, 1] + rp[:, 1:2] * lt[1, 1]
              + rp[:, 2:3] * lt[2, 1] + lt[3, 1] * ones)
        pz = (rp[:, 0:1] * lt[0, 2] + rp[:, 1:2] * lt[1, 2]
              + rp[:, 2:3] * lt[2, 2] + lt[3, 2] * ones)
        zc = jnp.maximum(pz, 1e-5)
        u = px / zc / 704.0 * _FW - 0.5
        v = py / zc / 256.0 * _FH - 0.5
        u = jnp.where(pz > 1e-5, u, -10.0)
        ax = jnp.maximum(1.0 - jnp.abs(u - pxf), 0.0)
        ay = jnp.maximum(1.0 - jnp.abs(v - pyf), 0.0)
        wcp = ay * ax
        samp = samp + jnp.dot(wcp.astype(jnp.bfloat16),
                              fcam.astype(jnp.bfloat16),
                              preferred_element_type=_F32)

    projt = projtref[...]
    pjb = pjbref[...]
    post = postref[...]
    psb = psbref[...]
    sf_pre = jnp.dot(samp[:, :128].astype(jnp.bfloat16), projt,
                     preferred_element_type=_F32) + pjb
    pos3 = samp[:, 128:131]
    d = rp - pos3
    nrm = jnp.sqrt(jnp.sum(d * d, axis=1, keepdims=True))
    wgt = jnp.exp(-_ALPHA * nrm * nrm)
    rn = [(rp[:, i:i + 1] - _LO[i]) * (1.0 / span[i]) for i in range(3)]
    pe = (rn[0] * post[0:1, :] + rn[1] * post[1:2, :]
          + rn[2] * post[2:3, :] + psb)
    sfall = sf_pre + pe
    sfwall = sf_pre * wgt
    sfacc = sfall[0:G]
    sfwacc = sfwall[0:G]
    for p in range(1, _NP):
        sfacc = sfacc + sfall[p * G:(p + 1) * G]
        sfwacc = sfwacc + sfwall[p * G:(p + 1) * G]
    sfref[0] = sfacc
    sfwref[0] = sfwacc


def _sampling(qf, posf, fcat, l2t, offt, ofb, projt, pjb, post, psb):
    full = lambda a: pl.BlockSpec(a.shape, lambda y: (0,) * a.ndim)
    return pl.pallas_call(
        _samp_body,
        grid=(_H,),
        in_specs=[
            pl.BlockSpec((1, _W, 128), lambda y: (y, 0, 0)),
            pl.BlockSpec((1, _W, 3), lambda y: (y, 0, 0)),
            full(fcat), full(l2t), full(offt), full(ofb),
            full(projt), full(pjb), full(post), full(psb),
        ],
        out_specs=[
            pl.BlockSpec((1, _W, 128), lambda y: (y, 0, 0)),
            pl.BlockSpec((1, _W, 128), lambda y: (y, 0, 0)),
        ],
        out_shape=[
            jax.ShapeDtypeStruct((_H, _W, 128), _F32),
            jax.ShapeDtypeStruct((_H, _W, 128), _F32),
        ],
    )(qf, posf, fcat, l2t, offt, ofb, projt, pjb, post, psb)


def _mid12_body(pref, w1ref, b1ref, w2ref, b2ref, oref):
    y = pl.program_id(0)
    rows = pref[pl.ds(y, 3)]
    acc = jnp.zeros((_W, 512), _F32) + b1ref[...]
    for dy in range(3):
        r = rows[dy]
        a = jnp.concatenate([r[dx:dx + _W] for dx in range(3)], axis=1)
        acc = acc + jnp.dot(a.astype(jnp.bfloat16), w1ref[dy],
                            preferred_element_type=_F32)
    acc = jax.nn.gelu(acc)
    acc = jnp.dot(acc.astype(jnp.bfloat16), w2ref[...],
                  preferred_element_type=_F32) + b2ref[...]
    oref[0] = jax.nn.gelu(acc)


def _mid12(p, w1, b1, w2, b2):
    full = lambda a: pl.BlockSpec(a.shape, lambda y: (0,) * a.ndim)
    return pl.pallas_call(
        _mid12_body,
        grid=(_H,),
        in_specs=[full(p), full(w1), full(b1), full(w2), full(b2)],
        out_specs=pl.BlockSpec((1, _W, 512), lambda y: (y, 0, 0)),
        out_shape=jax.ShapeDtypeStruct((_H, _W, 512), _F32),
    )(p, w1, b1, w2, b2)


def _mid3_body(pref, wref, bref, oref):
    y = pl.program_id(0)
    rows = pref[pl.ds(y, 3)]
    acc = jnp.zeros((_W, 128), _F32) + bref[...]
    for dy in range(3):
        r = rows[dy]
        a = jnp.concatenate([r[dx:dx + _W] for dx in range(3)], axis=1)
        acc = acc + jnp.dot(a.astype(jnp.bfloat16), wref[dy],
                            preferred_element_type=_F32)
    oref[0] = acc


def _mid3(p, w3, b3):
    full = lambda a: pl.BlockSpec(a.shape, lambda y: (0,) * a.ndim)
    return pl.pallas_call(
        _mid3_body,
        grid=(_H,),
        in_specs=[full(p), full(w3), full(b3)],
        out_specs=pl.BlockSpec((1, _W, 128), lambda y: (y, 0, 0)),
        out_shape=jax.ShapeDtypeStruct((_H, _W, 128), _F32),
    )(p, w3, b3)


def kernel(bev_query, bev_pos, mlvl_feats, lidar2img, in_conv_w, in_conv_b,
           off_w, off_b, proj_w, proj_b, pos_w, pos_b, mid_w1, mid_b1,
           mid_w2, mid_b2, mid_w3, mid_b3, out_w, out_b):
    bq = jnp.transpose(bev_query[0], (1, 2, 0))
    p1 = jnp.pad(bq, ((2, 2), (2, 2), (0, 0)))
    w_in = jnp.transpose(in_conv_w, (2, 3, 1, 0)).reshape(
        5, 5 * 128, 128).astype(jnp.bfloat16)
    r1 = _conv5_res(p1, w_in, in_conv_b[None])
    q1 = _inorm(r1.reshape(_Q, 128))

    fcat = jnp.transpose(mlvl_feats[0], (0, 2, 3, 1)).reshape(
        _NCAM, _NPIX, 132)
    l2t = jnp.transpose(lidar2img[0], (0, 2, 1))[:, :, :3]
    offt = off_w.reshape(12, 128).T
    sf, sfw = _sampling(q1.reshape(_H, _W, 128), bev_pos[0], fcat, l2t,
                        offt, off_b[None], proj_w.T.astype(jnp.bfloat16),
                        proj_b[None], pos_w.T, pos_b[None])

    cat = jnp.concatenate([sf, sfw], axis=2)
    p2 = jnp.pad(cat, ((1, 1), (1, 1), (0, 0)))
    w1 = jnp.transpose(mid_w1, (2, 3, 1, 0)).reshape(
        3, 3 * 256, 512).astype(jnp.bfloat16)
    w2 = mid_w2.reshape(512, 512).T.astype(jnp.bfloat16)
    mid2 = _mid12(p2, w1, mid_b1[None], w2, mid_b2[None])
    p3 = jnp.pad(mid2, ((1, 1), (1, 1), (0, 0)))
    w3 = jnp.transpose(mid_w3, (2, 3, 1, 0)).reshape(
        3, 3 * 512, 128).astype(jnp.bfloat16)
    mid = _mid3(p3, w3, mid_b3[None])

    q2 = _add_inorm(q1, mid.reshape(_Q, 128))
    p4 = jnp.pad(q2.reshape(_H, _W, 128), ((2, 2), (2, 2), (0, 0)))
    w_out = jnp.transpose(out_w, (2, 3, 1, 0)).reshape(
        5, 5 * 128, 128).astype(jnp.bfloat16)
    r2 = _conv5_res(p4, w_out, out_b[None])
    q3 = _inorm(r2.reshape(_Q, 128))
    return jnp.transpose(q3.reshape(_H, _W, 128), (2, 0, 1))[None]


# speedup vs baseline: 1.3682x; 1.0087x over previous
import jax
import jax.numpy as jnp
from jax.experimental import pallas as pl

_F32 = jnp.float32
_LO = (-51.2, -51.2, -5.0)
_HI = (51.2, 51.2, 3.0)
_EPS = 1e-6
_ALPHA = 0.1
_NP = 4
_FH, _FW = 16, 44
_NPIX = _FH * _FW
_PPAD = 768
_NCAM = 6
_H = _W = 100
_Q = _H * _W


def _conv5_res_body(pref, wref, bref, oref):
    y = pl.program_id(0)
    rows = pref[pl.ds(y, 5)]
    acc = rows[2][2:2 + _W]
    acc = acc + bref[...]
    for dy in range(5):
        r = rows[dy]
        a = jnp.concatenate([r[dx:dx + _W] for dx in range(5)], axis=1)
        acc = acc + jnp.dot(a.astype(jnp.bfloat16), wref[dy],
                            preferred_element_type=_F32)
    oref[0] = acc


def _conv5_res(p, w25, b):
    return pl.pallas_call(
        _conv5_res_body,
        grid=(_H,),
        in_specs=[
            pl.BlockSpec(p.shape, lambda y: (0, 0, 0)),
            pl.BlockSpec(w25.shape, lambda y: (0, 0, 0)),
            pl.BlockSpec(b.shape, lambda y: (0, 0)),
        ],
        out_specs=pl.BlockSpec((1, _W, 128), lambda y: (y, 0, 0)),
        out_shape=jax.ShapeDtypeStruct((_H, _W, 128), _F32),
    )(p, w25, b)


def _in_body(xref, oref):
    x = xref[...]
    m = jnp.mean(x, axis=0, keepdims=True)
    d = x - m
    v = jnp.mean(d * d, axis=0, keepdims=True)
    oref[...] = d * jax.lax.rsqrt(v + 1e-5)


def _inorm(x):
    return pl.pallas_call(
        _in_body,
        out_shape=jax.ShapeDtypeStruct(x.shape, _F32),
    )(x)


def _add_in_body(aref, bref, oref):
    x = aref[...] + bref[...]
    m = jnp.mean(x, axis=0, keepdims=True)
    d = x - m
    v = jnp.mean(d * d, axis=0, keepdims=True)
    oref[...] = d * jax.lax.rsqrt(v + 1e-5)


def _add_inorm(a, b):
    return pl.pallas_call(
        _add_in_body,
        out_shape=jax.ShapeDtypeStruct(a.shape, _F32),
    )(a, b)


def _samp_body(qref, posref, fref, lref, offtref, ofbref, projtref, pjbref,
               postref, psbref, sfref, sfwref):
    G = _W
    span = (_HI[0] - _LO[0], _HI[1] - _LO[1], _HI[2] - _LO[2])
    q = qref[0]
    offm = jax.nn.sigmoid(jnp.dot(q, offtref[...],
                                  preferred_element_type=_F32) + ofbref[...])
    pos = posref[0]
    refm = jnp.concatenate(
        [pos[:, i:i + 1] * span[i] + _LO[i] for i in range(3)], axis=1)

    iof = jax.lax.broadcasted_iota(jnp.int32, (_NP * G, _PPAD), 1).astype(_F32)
    pyf = jnp.floor(iof * (1.0 / _FW))
    pxf = iof - pyf * _FW

    rng = 0.25 + _EPS
    rngz = 4.0 + _EPS
    refp = []
    for p in range(_NP):
        o = offm[:, 3 * p:3 * p + 3]
        oxy = o[:, 0:2] * (2.0 * rng) - rng
        oz = o[:, 2:3] * (2.0 * rngz) - rngz
        refp.append(refm + jnp.concatenate([oxy, oz], axis=1))
    rp = jnp.concatenate(refp, axis=0)
    R = _NP * G

    wall = []
    ones = jnp.ones((R, 1), _F32)
    for cam in range(_NCAM):
        lt = lref[cam]
        px = (rp[:, 0:1] * lt[0, 0] + rp[:, 1:2] * lt[1, 0]
              + rp[:, 2:3] * lt[2, 0] + lt[3, 0] * ones)
        py = (rp[:, 0:1] * lt[0, 1] + rp[:, 1:2] * lt[1, 1]
              + rp[:, 2:3] * lt[2, 1] + lt[3, 1] * ones)
        pz = (rp[:, 0:1] * lt[0, 2] + rp[:, 1:2] * lt[1, 2]
              + rp[:, 2:3] * lt[2, 2] + lt[3, 2] * ones)
        zc = jnp.maximum(pz, 1e-5)
        u = px / zc / 704.0 * _FW - 0.5
        v = py / zc / 256.0 * _FH - 0.5
        u = jnp.where(pz > 1e-5, u, -10.0)
        ax = jnp.maximum(1.0 - jnp.abs(u - pxf), 0.0)
        ay = jnp.maximum(1.0 - jnp.abs(v - pyf), 0.0)
        wall.append((ay * ax).astype(jnp.bfloat16))
    wall = jnp.concatenate(wall, axis=1)
    samp = jnp.dot(wall, fref[...], preferred_element_type=_F32)

    projt = projtref[...]
    pjb = pjbref[...]
    post = postref[...]
    psb = psbref[...]
    sf_pre = jnp.dot(samp[:, :128].astype(jnp.bfloat16), projt,
                     preferred_element_type=_F32) + pjb
    pos3 = samp[:, 128:131]
    d = rp - pos3
    nrm = jnp.sqrt(jnp.sum(d * d, axis=1, keepdims=True))
    wgt = jnp.exp(-_ALPHA * nrm * nrm)
    rn = [(rp[:, i:i + 1] - _LO[i]) * (1.0 / span[i]) for i in range(3)]
    pe = (rn[0] * post[0:1, :] + rn[1] * post[1:2, :]
          + rn[2] * post[2:3, :] + psb)
    sfall = sf_pre + pe
    sfwall = sf_pre * wgt
    sfacc = sfall[0:G]
    sfwacc = sfwall[0:G]
    for p in range(1, _NP):
        sfacc = sfacc + sfall[p * G:(p + 1) * G]
        sfwacc = sfwacc + sfwall[p * G:(p + 1) * G]
    sfref[0] = sfacc
    sfwref[0] = sfwacc


def _sampling(qf, posf, fcat, l2t, offt, ofb, projt, pjb, post, psb):
    full = lambda a: pl.BlockSpec(a.shape, lambda y: (0,) * a.ndim)
    return pl.pallas_call(
        _samp_body,
        grid=(_H,),
        in_specs=[
            pl.BlockSpec((1, _W, 128), lambda y: (y, 0, 0)),
            pl.BlockSpec((1, _W, 3), lambda y: (y, 0, 0)),
            full(fcat), full(l2t), full(offt), full(ofb),
            full(projt), full(pjb), full(post), full(psb),
        ],
        out_specs=[
            pl.BlockSpec((1, _W, 128), lambda y: (y, 0, 0)),
            pl.BlockSpec((1, _W, 128), lambda y: (y, 0, 0)),
        ],
        out_shape=[
            jax.ShapeDtypeStruct((_H, _W, 128), _F32),
            jax.ShapeDtypeStruct((_H, _W, 128), _F32),
        ],
    )(qf, posf, fcat, l2t, offt, ofb, projt, pjb, post, psb)


def _mid12_body(pref, w1ref, b1ref, w2ref, b2ref, oref):
    y = pl.program_id(0)
    rows = pref[pl.ds(y, 3)]
    acc = jnp.zeros((_W, 512), _F32) + b1ref[...]
    for dy in range(3):
        r = rows[dy]
        a = jnp.concatenate([r[dx:dx + _W] for dx in range(3)], axis=1)
        acc = acc + jnp.dot(a.astype(jnp.bfloat16), w1ref[dy],
                            preferred_element_type=_F32)
    acc = jax.nn.gelu(acc)
    acc = jnp.dot(acc.astype(jnp.bfloat16), w2ref[...],
                  preferred_element_type=_F32) + b2ref[...]
    oref[0] = jax.nn.gelu(acc)


def _mid12(p, w1, b1, w2, b2):
    full = lambda a: pl.BlockSpec(a.shape, lambda y: (0,) * a.ndim)
    return pl.pallas_call(
        _mid12_body,
        grid=(_H,),
        in_specs=[full(p), full(w1), full(b1), full(w2), full(b2)],
        out_specs=pl.BlockSpec((1, _W, 512), lambda y: (y, 0, 0)),
        out_shape=jax.ShapeDtypeStruct((_H, _W, 512), _F32),
    )(p, w1, b1, w2, b2)


def _mid3_body(pref, wref, bref, oref):
    y = pl.program_id(0)
    rows = pref[pl.ds(y, 3)]
    acc = jnp.zeros((_W, 128), _F32) + bref[...]
    for dy in range(3):
        r = rows[dy]
        a = jnp.concatenate([r[dx:dx + _W] for dx in range(3)], axis=1)
        acc = acc + jnp.dot(a.astype(jnp.bfloat16), wref[dy],
                            preferred_element_type=_F32)
    oref[0] = acc


def _mid3(p, w3, b3):
    full = lambda a: pl.BlockSpec(a.shape, lambda y: (0,) * a.ndim)
    return pl.pallas_call(
        _mid3_body,
        grid=(_H,),
        in_specs=[full(p), full(w3), full(b3)],
        out_specs=pl.BlockSpec((1, _W, 128), lambda y: (y, 0, 0)),
        out_shape=jax.ShapeDtypeStruct((_H, _W, 128), _F32),
    )(p, w3, b3)


def kernel(bev_query, bev_pos, mlvl_feats, lidar2img, in_conv_w, in_conv_b,
           off_w, off_b, proj_w, proj_b, pos_w, pos_b, mid_w1, mid_b1,
           mid_w2, mid_b2, mid_w3, mid_b3, out_w, out_b):
    bq = jnp.transpose(bev_query[0], (1, 2, 0))
    p1 = jnp.pad(bq, ((2, 2), (2, 2), (0, 0)))
    w_in = jnp.transpose(in_conv_w, (2, 3, 1, 0)).reshape(
        5, 5 * 128, 128).astype(jnp.bfloat16)
    r1 = _conv5_res(p1, w_in, in_conv_b[None])
    q1 = _inorm(r1.reshape(_Q, 128))

    fcat = jnp.transpose(mlvl_feats[0], (0, 2, 3, 1)).reshape(
        _NCAM, _NPIX, 132)
    fcat = jnp.pad(fcat, ((0, 0), (0, _PPAD - _NPIX), (0, 0))).reshape(
        _NCAM * _PPAD, 132).astype(jnp.bfloat16)
    l2t = jnp.transpose(lidar2img[0], (0, 2, 1))[:, :, :3]
    offt = off_w.reshape(12, 128).T
    sf, sfw = _sampling(q1.reshape(_H, _W, 128), bev_pos[0], fcat, l2t,
                        offt, off_b[None], proj_w.T.astype(jnp.bfloat16),
                        proj_b[None], pos_w.T, pos_b[None])

    cat = jnp.concatenate([sf, sfw], axis=2)
    p2 = jnp.pad(cat, ((1, 1), (1, 1), (0, 0)))
    w1 = jnp.transpose(mid_w1, (2, 3, 1, 0)).reshape(
        3, 3 * 256, 512).astype(jnp.bfloat16)
    w2 = mid_w2.reshape(512, 512).T.astype(jnp.bfloat16)
    mid2 = _mid12(p2, w1, mid_b1[None], w2, mid_b2[None])
    p3 = jnp.pad(mid2, ((1, 1), (1, 1), (0, 0)))
    w3 = jnp.transpose(mid_w3, (2, 3, 1, 0)).reshape(
        3, 3 * 512, 128).astype(jnp.bfloat16)
    mid = _mid3(p3, w3, mid_b3[None])

    q2 = _add_inorm(q1, mid.reshape(_Q, 128))
    p4 = jnp.pad(q2.reshape(_H, _W, 128), ((2, 2), (2, 2), (0, 0)))
    w_out = jnp.transpose(out_w, (2, 3, 1, 0)).reshape(
        5, 5 * 128, 128).astype(jnp.bfloat16)
    r2 = _conv5_res(p4, w_out, out_b[None])
    q3 = _inorm(r2.reshape(_Q, 128))
    return jnp.transpose(q3.reshape(_H, _W, 128), (2, 0, 1))[None]


# speedup vs baseline: 1.4944x; 1.0922x over previous
import jax
import jax.numpy as jnp
from jax.experimental import pallas as pl

_F32 = jnp.float32
_LO = (-51.2, -51.2, -5.0)
_HI = (51.2, 51.2, 3.0)
_EPS = 1e-6
_ALPHA = 0.1
_NP = 4
_FH, _FW = 16, 44
_NPIX = _FH * _FW
_PPAD = 768
_RB = 2
_NCAM = 6
_H = _W = 100
_Q = _H * _W


def _conv5_res_body(pref, wref, bref, oref):
    y = pl.program_id(0)
    rows = pref[pl.ds(y, 5)]
    acc = rows[2][2:2 + _W]
    acc = acc + bref[...]
    for dy in range(5):
        r = rows[dy]
        a = jnp.concatenate([r[dx:dx + _W] for dx in range(5)], axis=1)
        acc = acc + jnp.dot(a.astype(jnp.bfloat16), wref[dy],
                            preferred_element_type=_F32)
    oref[0] = acc


def _conv5_res(p, w25, b):
    return pl.pallas_call(
        _conv5_res_body,
        grid=(_H,),
        in_specs=[
            pl.BlockSpec(p.shape, lambda y: (0, 0, 0)),
            pl.BlockSpec(w25.shape, lambda y: (0, 0, 0)),
            pl.BlockSpec(b.shape, lambda y: (0, 0)),
        ],
        out_specs=pl.BlockSpec((1, _W, 128), lambda y: (y, 0, 0)),
        out_shape=jax.ShapeDtypeStruct((_H, _W, 128), _F32),
    )(p, w25, b)


def _in_body(xref, oref):
    x = xref[...]
    m = jnp.mean(x, axis=0, keepdims=True)
    d = x - m
    v = jnp.mean(d * d, axis=0, keepdims=True)
    oref[...] = d * jax.lax.rsqrt(v + 1e-5)


def _inorm(x):
    return pl.pallas_call(
        _in_body,
        out_shape=jax.ShapeDtypeStruct(x.shape, _F32),
    )(x)


def _add_in_body(aref, bref, oref):
    x = aref[...] + bref[...]
    m = jnp.mean(x, axis=0, keepdims=True)
    d = x - m
    v = jnp.mean(d * d, axis=0, keepdims=True)
    oref[...] = d * jax.lax.rsqrt(v + 1e-5)


def _add_inorm(a, b):
    return pl.pallas_call(
        _add_in_body,
        out_shape=jax.ShapeDtypeStruct(a.shape, _F32),
    )(a, b)


def _samp_body(qref, posref, fref, lref, offtref, ofbref, projtref, pjbref,
               postref, psbref, sfref, sfwref):
    G = _RB * _W
    span = (_HI[0] - _LO[0], _HI[1] - _LO[1], _HI[2] - _LO[2])
    q = qref[...].reshape(G, 128)
    offm = jax.nn.sigmoid(jnp.dot(q, offtref[...],
                                  preferred_element_type=_F32) + ofbref[...])
    pos = posref[...].reshape(G, 3)
    refm = jnp.concatenate(
        [pos[:, i:i + 1] * span[i] + _LO[i] for i in range(3)], axis=1)

    iof = jax.lax.broadcasted_iota(jnp.int32, (_NP * G, _PPAD), 1).astype(_F32)
    pyf = jnp.floor(iof * (1.0 / _FW))
    pxf = iof - pyf * _FW

    rng = 0.25 + _EPS
    rngz = 4.0 + _EPS
    refp = []
    for p in range(_NP):
        o = offm[:, 3 * p:3 * p + 3]
        oxy = o[:, 0:2] * (2.0 * rng) - rng
        oz = o[:, 2:3] * (2.0 * rngz) - rngz
        refp.append(refm + jnp.concatenate([oxy, oz], axis=1))
    rp = jnp.concatenate(refp, axis=0)
    R = _NP * G

    wall = []
    ones = jnp.ones((R, 1), _F32)
    for cam in range(_NCAM):
        lt = lref[cam]
        px = (rp[:, 0:1] * lt[0, 0] + rp[:, 1:2] * lt[1, 0]
              + rp[:, 2:3] * lt[2, 0] + lt[3, 0] * ones)
        py = (rp[:, 0:1] * lt[0, 1] + rp[:, 1:2] * lt[1, 1]
              + rp[:, 2:3] * lt[2, 1] + lt[3, 1] * ones)
        pz = (rp[:, 0:1] * lt[0, 2] + rp[:, 1:2] * lt[1, 2]
              + rp[:, 2:3] * lt[2, 2] + lt[3, 2] * ones)
        zc = jnp.maximum(pz, 1e-5)
        u = px / zc / 704.0 * _FW - 0.5
        v = py / zc / 256.0 * _FH - 0.5
        u = jnp.where(pz > 1e-5, u, -10.0)
        ax = jnp.maximum(1.0 - jnp.abs(u - pxf), 0.0)
        ay = jnp.maximum(1.0 - jnp.abs(v - pyf), 0.0)
        wall.append((ay * ax).astype(jnp.bfloat16))
    wall = jnp.concatenate(wall, axis=1)
    samp = jnp.dot(wall, fref[...], preferred_element_type=_F32)

    projt = projtref[...]
    pjb = pjbref[...]
    post = postref[...]
    psb = psbref[...]
    sf_pre = jnp.dot(samp[:, :128].astype(jnp.bfloat16), projt,
                     preferred_element_type=_F32) + pjb
    pos3 = samp[:, 128:131]
    d = rp - pos3
    nrm = jnp.sqrt(jnp.sum(d * d, axis=1, keepdims=True))
    wgt = jnp.exp(-_ALPHA * nrm * nrm)
    rn = [(rp[:, i:i + 1] - _LO[i]) * (1.0 / span[i]) for i in range(3)]
    pe = (rn[0] * post[0:1, :] + rn[1] * post[1:2, :]
          + rn[2] * post[2:3, :] + psb)
    sfall = sf_pre + pe
    sfwall = sf_pre * wgt
    sfacc = sfall[0:G]
    sfwacc = sfwall[0:G]
    for p in range(1, _NP):
        sfacc = sfacc + sfall[p * G:(p + 1) * G]
        sfwacc = sfwacc + sfwall[p * G:(p + 1) * G]
    sfref[...] = sfacc.reshape(_RB, _W, 128)
    sfwref[...] = sfwacc.reshape(_RB, _W, 128)


def _sampling(qf, posf, fcat, l2t, offt, ofb, projt, pjb, post, psb):
    full = lambda a: pl.BlockSpec(a.shape, lambda y: (0,) * a.ndim)
    return pl.pallas_call(
        _samp_body,
        grid=(_H // _RB,),
        in_specs=[
            pl.BlockSpec((_RB, _W, 128), lambda y: (y, 0, 0)),
            pl.BlockSpec((_RB, _W, 3), lambda y: (y, 0, 0)),
            full(fcat), full(l2t), full(offt), full(ofb),
            full(projt), full(pjb), full(post), full(psb),
        ],
        out_specs=[
            pl.BlockSpec((_RB, _W, 128), lambda y: (y, 0, 0)),
            pl.BlockSpec((_RB, _W, 128), lambda y: (y, 0, 0)),
        ],
        out_shape=[
            jax.ShapeDtypeStruct((_H, _W, 128), _F32),
            jax.ShapeDtypeStruct((_H, _W, 128), _F32),
        ],
    )(qf, posf, fcat, l2t, offt, ofb, projt, pjb, post, psb)


def _mid12_body(pref, w1ref, b1ref, w2ref, b2ref, oref):
    y = pl.program_id(0)
    rows = pref[pl.ds(y, 3)]
    acc = jnp.zeros((_W, 512), _F32) + b1ref[...]
    for dy in range(3):
        r = rows[dy]
        a = jnp.concatenate([r[dx:dx + _W] for dx in range(3)], axis=1)
        acc = acc + jnp.dot(a.astype(jnp.bfloat16), w1ref[dy],
                            preferred_element_type=_F32)
    acc = jax.nn.gelu(acc)
    acc = jnp.dot(acc.astype(jnp.bfloat16), w2ref[...],
                  preferred_element_type=_F32) + b2ref[...]
    oref[0] = jax.nn.gelu(acc)


def _mid12(p, w1, b1, w2, b2):
    full = lambda a: pl.BlockSpec(a.shape, lambda y: (0,) * a.ndim)
    return pl.pallas_call(
        _mid12_body,
        grid=(_H,),
        in_specs=[full(p), full(w1), full(b1), full(w2), full(b2)],
        out_specs=pl.BlockSpec((1, _W, 512), lambda y: (y, 0, 0)),
        out_shape=jax.ShapeDtypeStruct((_H, _W, 512), _F32),
    )(p, w1, b1, w2, b2)


def _mid3_body(pref, wref, bref, oref):
    y = pl.program_id(0)
    rows = pref[pl.ds(y, 3)]
    acc = jnp.zeros((_W, 128), _F32) + bref[...]
    for dy in range(3):
        r = rows[dy]
        a = jnp.concatenate([r[dx:dx + _W] for dx in range(3)], axis=1)
        acc = acc + jnp.dot(a.astype(jnp.bfloat16), wref[dy],
                            preferred_element_type=_F32)
    oref[0] = acc


def _mid3(p, w3, b3):
    full = lambda a: pl.BlockSpec(a.shape, lambda y: (0,) * a.ndim)
    return pl.pallas_call(
        _mid3_body,
        grid=(_H,),
        in_specs=[full(p), full(w3), full(b3)],
        out_specs=pl.BlockSpec((1, _W, 128), lambda y: (y, 0, 0)),
        out_shape=jax.ShapeDtypeStruct((_H, _W, 128), _F32),
    )(p, w3, b3)


def kernel(bev_query, bev_pos, mlvl_feats, lidar2img, in_conv_w, in_conv_b,
           off_w, off_b, proj_w, proj_b, pos_w, pos_b, mid_w1, mid_b1,
           mid_w2, mid_b2, mid_w3, mid_b3, out_w, out_b):
    bq = jnp.transpose(bev_query[0], (1, 2, 0))
    p1 = jnp.pad(bq, ((2, 2), (2, 2), (0, 0)))
    w_in = jnp.transpose(in_conv_w, (2, 3, 1, 0)).reshape(
        5, 5 * 128, 128).astype(jnp.bfloat16)
    r1 = _conv5_res(p1, w_in, in_conv_b[None])
    q1 = _inorm(r1.reshape(_Q, 128))

    fcat = jnp.transpose(mlvl_feats[0], (0, 2, 3, 1)).reshape(
        _NCAM, _NPIX, 132)
    fcat = jnp.pad(fcat, ((0, 0), (0, _PPAD - _NPIX), (0, 0))).reshape(
        _NCAM * _PPAD, 132).astype(jnp.bfloat16)
    l2t = jnp.transpose(lidar2img[0], (0, 2, 1))[:, :, :3]
    offt = off_w.reshape(12, 128).T
    sf, sfw = _sampling(q1.reshape(_H, _W, 128), bev_pos[0], fcat, l2t,
                        offt, off_b[None], proj_w.T.astype(jnp.bfloat16),
                        proj_b[None], pos_w.T, pos_b[None])

    cat = jnp.concatenate([sf, sfw], axis=2)
    p2 = jnp.pad(cat, ((1, 1), (1, 1), (0, 0)))
    w1 = jnp.transpose(mid_w1, (2, 3, 1, 0)).reshape(
        3, 3 * 256, 512).astype(jnp.bfloat16)
    w2 = mid_w2.reshape(512, 512).T.astype(jnp.bfloat16)
    mid2 = _mid12(p2, w1, mid_b1[None], w2, mid_b2[None])
    p3 = jnp.pad(mid2, ((1, 1), (1, 1), (0, 0)))
    w3 = jnp.transpose(mid_w3, (2, 3, 1, 0)).reshape(
        3, 3 * 512, 128).astype(jnp.bfloat16)
    mid = _mid3(p3, w3, mid_b3[None])

    q2 = _add_inorm(q1, mid.reshape(_Q, 128))
    p4 = jnp.pad(q2.reshape(_H, _W, 128), ((2, 2), (2, 2), (0, 0)))
    w_out = jnp.transpose(out_w, (2, 3, 1, 0)).reshape(
        5, 5 * 128, 128).astype(jnp.bfloat16)
    r2 = _conv5_res(p4, w_out, out_b[None])
    q3 = _inorm(r2.reshape(_Q, 128))
    return jnp.transpose(q3.reshape(_H, _W, 128), (2, 0, 1))[None]


# speedup vs baseline: 1.5657x; 1.0477x over previous
import jax
import jax.numpy as jnp
from jax.experimental import pallas as pl

_F32 = jnp.float32
_LO = (-51.2, -51.2, -5.0)
_HI = (51.2, 51.2, 3.0)
_EPS = 1e-6
_ALPHA = 0.1
_NP = 4
_FH, _FW = 16, 44
_NPIX = _FH * _FW
_PPAD = 768
_RB = 4
_NCAM = 6
_H = _W = 100
_Q = _H * _W


def _conv5_res_body(pref, wref, bref, oref):
    y = pl.program_id(0)
    rows = pref[pl.ds(y, 5)]
    acc = rows[2][2:2 + _W]
    acc = acc + bref[...]
    for dy in range(5):
        r = rows[dy]
        a = jnp.concatenate([r[dx:dx + _W] for dx in range(5)], axis=1)
        acc = acc + jnp.dot(a.astype(jnp.bfloat16), wref[dy],
                            preferred_element_type=_F32)
    oref[0] = acc


def _conv5_res(p, w25, b):
    return pl.pallas_call(
        _conv5_res_body,
        grid=(_H,),
        in_specs=[
            pl.BlockSpec(p.shape, lambda y: (0, 0, 0)),
            pl.BlockSpec(w25.shape, lambda y: (0, 0, 0)),
            pl.BlockSpec(b.shape, lambda y: (0, 0)),
        ],
        out_specs=pl.BlockSpec((1, _W, 128), lambda y: (y, 0, 0)),
        out_shape=jax.ShapeDtypeStruct((_H, _W, 128), _F32),
    )(p, w25, b)


def _in_body(xref, oref):
    x = xref[...]
    m = jnp.mean(x, axis=0, keepdims=True)
    d = x - m
    v = jnp.mean(d * d, axis=0, keepdims=True)
    oref[...] = d * jax.lax.rsqrt(v + 1e-5)


def _inorm(x):
    return pl.pallas_call(
        _in_body,
        out_shape=jax.ShapeDtypeStruct(x.shape, _F32),
    )(x)


def _add_in_body(aref, bref, oref):
    x = aref[...] + bref[...]
    m = jnp.mean(x, axis=0, keepdims=True)
    d = x - m
    v = jnp.mean(d * d, axis=0, keepdims=True)
    oref[...] = d * jax.lax.rsqrt(v + 1e-5)


def _add_inorm(a, b):
    return pl.pallas_call(
        _add_in_body,
        out_shape=jax.ShapeDtypeStruct(a.shape, _F32),
    )(a, b)


def _samp_body(qref, posref, fref, lref, offtref, ofbref, projtref, pjbref,
               postref, psbref, sfref, sfwref):
    G = _RB * _W
    span = (_HI[0] - _LO[0], _HI[1] - _LO[1], _HI[2] - _LO[2])
    q = qref[...].reshape(G, 128)
    offm = jax.nn.sigmoid(jnp.dot(q, offtref[...],
                                  preferred_element_type=_F32) + ofbref[...])
    pos = posref[...].reshape(G, 3)
    refm = jnp.concatenate(
        [pos[:, i:i + 1] * span[i] + _LO[i] for i in range(3)], axis=1)

    iof = jax.lax.broadcasted_iota(jnp.int32, (_NP * G, _PPAD), 1).astype(_F32)
    pyf = jnp.floor(iof * (1.0 / _FW))
    pxf = iof - pyf * _FW

    rng = 0.25 + _EPS
    rngz = 4.0 + _EPS
    refp = []
    for p in range(_NP):
        o = offm[:, 3 * p:3 * p + 3]
        oxy = o[:, 0:2] * (2.0 * rng) - rng
        oz = o[:, 2:3] * (2.0 * rngz) - rngz
        refp.append(refm + jnp.concatenate([oxy, oz], axis=1))
    rp = jnp.concatenate(refp, axis=0)
    R = _NP * G

    wall = []
    ones = jnp.ones((R, 1), _F32)
    for cam in range(_NCAM):
        lt = lref[cam]
        px = (rp[:, 0:1] * lt[0, 0] + rp[:, 1:2] * lt[1, 0]
              + rp[:, 2:3] * lt[2, 0] + lt[3, 0] * ones)
        py = (rp[:, 0:1] * lt[0, 1] + rp[:, 1:2] * lt[1, 1]
              + rp[:, 2:3] * lt[2, 1] + lt[3, 1] * ones)
        pz = (rp[:, 0:1] * lt[0, 2] + rp[:, 1:2] * lt[1, 2]
              + rp[:, 2:3] * lt[2, 2] + lt[3, 2] * ones)
        zc = jnp.maximum(pz, 1e-5)
        u = px / zc / 704.0 * _FW - 0.5
        v = py / zc / 256.0 * _FH - 0.5
        u = jnp.where(pz > 1e-5, u, -10.0)
        ax = jnp.maximum(1.0 - jnp.abs(u - pxf), 0.0)
        ay = jnp.maximum(1.0 - jnp.abs(v - pyf), 0.0)
        wall.append((ay * ax).astype(jnp.bfloat16))
    wall = jnp.concatenate(wall, axis=1)
    samp = jnp.dot(wall, fref[...], preferred_element_type=_F32)

    projt = projtref[...]
    pjb = pjbref[...]
    post = postref[...]
    psb = psbref[...]
    sf_pre = jnp.dot(samp[:, :128].astype(jnp.bfloat16), projt,
                     preferred_element_type=_F32) + pjb
    pos3 = samp[:, 128:131]
    d = rp - pos3
    nrm = jnp.sqrt(jnp.sum(d * d, axis=1, keepdims=True))
    wgt = jnp.exp(-_ALPHA * nrm * nrm)
    rn = [(rp[:, i:i + 1] - _LO[i]) * (1.0 / span[i]) for i in range(3)]
    pe = (rn[0] * post[0:1, :] + rn[1] * post[1:2, :]
          + rn[2] * post[2:3, :] + psb)
    sfall = sf_pre + pe
    sfwall = sf_pre * wgt
    sfacc = sfall[0:G]
    sfwacc = sfwall[0:G]
    for p in range(1, _NP):
        sfacc = sfacc + sfall[p * G:(p + 1) * G]
        sfwacc = sfwacc + sfwall[p * G:(p + 1) * G]
    sfref[...] = sfacc.reshape(_RB, _W, 128)
    sfwref[...] = sfwacc.reshape(_RB, _W, 128)


def _sampling(qf, posf, fcat, l2t, offt, ofb, projt, pjb, post, psb):
    full = lambda a: pl.BlockSpec(a.shape, lambda y: (0,) * a.ndim)
    return pl.pallas_call(
        _samp_body,
        grid=(_H // _RB,),
        in_specs=[
            pl.BlockSpec((_RB, _W, 128), lambda y: (y, 0, 0)),
            pl.BlockSpec((_RB, _W, 3), lambda y: (y, 0, 0)),
            full(fcat), full(l2t), full(offt), full(ofb),
            full(projt), full(pjb), full(post), full(psb),
        ],
        out_specs=[
            pl.BlockSpec((_RB, _W, 128), lambda y: (y, 0, 0)),
            pl.BlockSpec((_RB, _W, 128), lambda y: (y, 0, 0)),
        ],
        out_shape=[
            jax.ShapeDtypeStruct((_H, _W, 128), _F32),
            jax.ShapeDtypeStruct((_H, _W, 128), _F32),
        ],
    )(qf, posf, fcat, l2t, offt, ofb, projt, pjb, post, psb)


def _mid12_body(pref, w1ref, b1ref, w2ref, b2ref, oref):
    y = pl.program_id(0)
    rows = pref[pl.ds(y, 3)]
    acc = jnp.zeros((_W, 512), _F32) + b1ref[...]
    for dy in range(3):
        r = rows[dy]
        a = jnp.concatenate([r[dx:dx + _W] for dx in range(3)], axis=1)
        acc = acc + jnp.dot(a.astype(jnp.bfloat16), w1ref[dy],
                            preferred_element_type=_F32)
    acc = jax.nn.gelu(acc)
    acc = jnp.dot(acc.astype(jnp.bfloat16), w2ref[...],
                  preferred_element_type=_F32) + b2ref[...]
    oref[0] = jax.nn.gelu(acc)


def _mid12(p, w1, b1, w2, b2):
    full = lambda a: pl.BlockSpec(a.shape, lambda y: (0,) * a.ndim)
    return pl.pallas_call(
        _mid12_body,
        grid=(_H,),
        in_specs=[full(p), full(w1), full(b1), full(w2), full(b2)],
        out_specs=pl.BlockSpec((1, _W, 512), lambda y: (y, 0, 0)),
        out_shape=jax.ShapeDtypeStruct((_H, _W, 512), _F32),
    )(p, w1, b1, w2, b2)


def _mid3_body(pref, wref, bref, oref):
    y = pl.program_id(0)
    rows = pref[pl.ds(y, 3)]
    acc = jnp.zeros((_W, 128), _F32) + bref[...]
    for dy in range(3):
        r = rows[dy]
        a = jnp.concatenate([r[dx:dx + _W] for dx in range(3)], axis=1)
        acc = acc + jnp.dot(a.astype(jnp.bfloat16), wref[dy],
                            preferred_element_type=_F32)
    oref[0] = acc


def _mid3(p, w3, b3):
    full = lambda a: pl.BlockSpec(a.shape, lambda y: (0,) * a.ndim)
    return pl.pallas_call(
        _mid3_body,
        grid=(_H,),
        in_specs=[full(p), full(w3), full(b3)],
        out_specs=pl.BlockSpec((1, _W, 128), lambda y: (y, 0, 0)),
        out_shape=jax.ShapeDtypeStruct((_H, _W, 128), _F32),
    )(p, w3, b3)


def kernel(bev_query, bev_pos, mlvl_feats, lidar2img, in_conv_w, in_conv_b,
           off_w, off_b, proj_w, proj_b, pos_w, pos_b, mid_w1, mid_b1,
           mid_w2, mid_b2, mid_w3, mid_b3, out_w, out_b):
    bq = jnp.transpose(bev_query[0], (1, 2, 0))
    p1 = jnp.pad(bq, ((2, 2), (2, 2), (0, 0)))
    w_in = jnp.transpose(in_conv_w, (2, 3, 1, 0)).reshape(
        5, 5 * 128, 128).astype(jnp.bfloat16)
    r1 = _conv5_res(p1, w_in, in_conv_b[None])
    q1 = _inorm(r1.reshape(_Q, 128))

    fcat = jnp.transpose(mlvl_feats[0], (0, 2, 3, 1)).reshape(
        _NCAM, _NPIX, 132)
    fcat = jnp.pad(fcat, ((0, 0), (0, _PPAD - _NPIX), (0, 0))).reshape(
        _NCAM * _PPAD, 132).astype(jnp.bfloat16)
    l2t = jnp.transpose(lidar2img[0], (0, 2, 1))[:, :, :3]
    offt = off_w.reshape(12, 128).T
    sf, sfw = _sampling(q1.reshape(_H, _W, 128), bev_pos[0], fcat, l2t,
                        offt, off_b[None], proj_w.T.astype(jnp.bfloat16),
                        proj_b[None], pos_w.T, pos_b[None])

    cat = jnp.concatenate([sf, sfw], axis=2)
    p2 = jnp.pad(cat, ((1, 1), (1, 1), (0, 0)))
    w1 = jnp.transpose(mid_w1, (2, 3, 1, 0)).reshape(
        3, 3 * 256, 512).astype(jnp.bfloat16)
    w2 = mid_w2.reshape(512, 512).T.astype(jnp.bfloat16)
    mid2 = _mid12(p2, w1, mid_b1[None], w2, mid_b2[None])
    p3 = jnp.pad(mid2, ((1, 1), (1, 1), (0, 0)))
    w3 = jnp.transpose(mid_w3, (2, 3, 1, 0)).reshape(
        3, 3 * 512, 128).astype(jnp.bfloat16)
    mid = _mid3(p3, w3, mid_b3[None])

    q2 = _add_inorm(q1, mid.reshape(_Q, 128))
    p4 = jnp.pad(q2.reshape(_H, _W, 128), ((2, 2), (2, 2), (0, 0)))
    w_out = jnp.transpose(out_w, (2, 3, 1, 0)).reshape(
        5, 5 * 128, 128).astype(jnp.bfloat16)
    r2 = _conv5_res(p4, w_out, out_b[None])
    q3 = _inorm(r2.reshape(_Q, 128))
    return jnp.transpose(q3.reshape(_H, _W, 128), (2, 0, 1))[None]
